# baseline (speedup 1.0000x reference)
import functools

import jax
import jax.numpy as jnp
from jax import lax
from jax.experimental import pallas as pl
from jax.experimental.pallas import tpu as pltpu
from jax.experimental.pallas import tpu_sc as plsc

N_NODES = 50000
BN = 1000

NC = 2
NS = 16
LANES = 16
E_EDGES = 800000
E_W = E_EDGES // (NC * NS)
K = 128
NFULL = E_W // K
TAIL = E_W - NFULL * K


def _sc_msg_body(h_hbm, g_hbm, b_hbm, src_hbm, dst_hbm, msg_hbm,
                 src_b, dst_b, h_buf, g_buf, b_buf, sem, *, C):
    cid = lax.axis_index("c")
    sid = lax.axis_index("s")
    nv = C // LANES
    e0 = (sid * NC + cid) * E_W

    def do_batch(gpos, nreal):
        d1 = pltpu.async_copy(h_hbm.at[src_b], h_buf, sem)
        d2 = pltpu.async_copy(g_hbm.at[dst_b], g_buf, sem)
        d3 = pltpu.async_copy(b_hbm.at[dst_b], b_buf, sem)
        d1.wait(); d2.wait(); d3.wait()

        def row_body(r, _w):
            for c in range(nv):
                sl = pl.ds(c * LANES, LANES)
                h_buf[r, sl] = jnp.maximum(
                    g_buf[r, sl] * h_buf[r, sl] + b_buf[r, sl], 0.0)
            return _w
        lax.fori_loop(0, K, row_body, 0)
        if nreal == K:
            pltpu.sync_copy(h_buf, msg_hbm.at[pl.ds(gpos, K)])
        else:
            pltpu.sync_copy(h_buf.at[pl.ds(0, nreal)],
                            msg_hbm.at[pl.ds(gpos, nreal)])

    def batch_body(bj, _u):
        gpos = e0 + bj * K
        pltpu.sync_copy(src_hbm.at[pl.ds(gpos, K)], src_b)
        pltpu.sync_copy(dst_hbm.at[pl.ds(gpos, K)], dst_b)
        do_batch(gpos, K)
        return _u
    lax.fori_loop(0, NFULL, batch_body, 0)

    gpos = e0 + NFULL * K
    pltpu.sync_copy(src_hbm.at[pl.ds(gpos, TAIL)], src_b.at[pl.ds(0, TAIL)])
    pltpu.sync_copy(dst_hbm.at[pl.ds(gpos, TAIL)], dst_b.at[pl.ds(0, TAIL)])
    do_batch(gpos, TAIL)


def _sc_msg(h, gamma, beta, src, dst, C):
    mesh = plsc.VectorSubcoreMesh(core_axis_name="c", subcore_axis_name="s")
    kern = pl.kernel(
        functools.partial(_sc_msg_body, C=C),
        out_type=jax.ShapeDtypeStruct((E_EDGES, C), jnp.float32),
        mesh=mesh,
        scratch_types=[
            pltpu.VMEM((K,), jnp.int32),
            pltpu.VMEM((K,), jnp.int32),
            pltpu.VMEM((K, C), jnp.float32),
            pltpu.VMEM((K, C), jnp.float32),
            pltpu.VMEM((K, C), jnp.float32),
            pltpu.SemaphoreType.DMA,
        ],
    )
    return kern(h, gamma, beta, src, dst)


def _dense_body(x_ref, w_ref, b_ref, base_ref, gamma_ref, beta_ref, h_ref, *, cp: int):
    x = x_ref[...]
    z = jnp.dot(x, w_ref[...], preferred_element_type=jnp.float32) + b_ref[...]
    s = z[:, 0:cp]
    fs_beta = z[:, cp:2 * cp]
    fs_gamma = z[:, 2 * cp:3 * cp]
    f_beta = z[:, 3 * cp:4 * cp]
    f_gamma = z[:, 4 * cp:5 * cp]
    lin = z[:, 5 * cp:6 * cp]
    res = z[:, 6 * cp:7 * cp]
    base_ref[...] = jax.nn.relu(fs_gamma * s + fs_beta) + res
    gamma_ref[...] = f_gamma
    beta_ref[...] = f_beta
    h_ref[...] = lin


def _dense_layer(x, skip_W, fskip_W, fskip_b, film_W, film_b, lin_W, L_W, L_b,
                 out_ch):
    n, f = x.shape
    cp = ((out_ch + 127) // 128) * 128
    pad_c = cp - out_ch

    def padc(w):
        return jnp.pad(w, ((0, 0), (0, pad_c))) if pad_c else w

    fs_b = fskip_W[:, :out_ch]
    fs_g = fskip_W[:, out_ch:]
    f_b = film_W[:, :out_ch]
    f_g = film_W[:, out_ch:]
    wcat = jnp.concatenate(
        [padc(skip_W), padc(fs_b), padc(fs_g), padc(f_b), padc(f_g),
         padc(lin_W), padc(L_W)], axis=1)

    def padb(b):
        return jnp.pad(b, (0, pad_c)) if pad_c else b

    zero = jnp.zeros((cp,), jnp.float32)
    bcat = jnp.concatenate(
        [zero, padb(fskip_b[:out_ch]), padb(fskip_b[out_ch:]),
         padb(film_b[:out_ch]), padb(film_b[out_ch:]), zero, padb(L_b)])
    bcat = bcat.reshape(1, 7 * cp)

    fp = ((f + 7) // 8) * 8
    if fp != f:
        x = jnp.pad(x, ((0, 0), (0, fp - f)))
        wcat = jnp.pad(wcat, ((0, fp - f), (0, 0)))

    grid = (n // BN,)
    out_shape = [jax.ShapeDtypeStruct((n, cp), jnp.float32)] * 4
    outs = pl.pallas_call(
        functools.partial(_dense_body, cp=cp),
        grid=grid,
        in_specs=[
            pl.BlockSpec((BN, fp), lambda i: (i, 0)),
            pl.BlockSpec((fp, 7 * cp), lambda i: (0, 0)),
            pl.BlockSpec((1, 7 * cp), lambda i: (0, 0)),
        ],
        out_specs=[pl.BlockSpec((BN, cp), lambda i: (i, 0))] * 4,
        out_shape=out_shape,
    )(x, wcat, bcat)
    return outs


def _combine_body(base_ref, agg_ref, cnt_ref, out_ref, *, act):
    cnt = jnp.maximum(cnt_ref[...], 1.0)
    v = base_ref[...] + agg_ref[...] / cnt
    if act:
        v = jnp.where(v > 0, v, jnp.exp(jnp.minimum(v, 0.0)) - 1.0)
    out_ref[...] = v


def _combine(base, agg_sum, cnt, act):
    n, cp = base.shape
    grid = (n // BN,)
    return pl.pallas_call(
        functools.partial(_combine_body, act=act),
        grid=grid,
        in_specs=[
            pl.BlockSpec((BN, cp), lambda i: (i, 0)),
            pl.BlockSpec((BN, cp), lambda i: (i, 0)),
            pl.BlockSpec((BN, 1), lambda i: (i, 0)),
        ],
        out_specs=pl.BlockSpec((BN, cp), lambda i: (i, 0)),
        out_shape=jax.ShapeDtypeStruct((n, cp), jnp.float32),
    )(base, agg_sum, cnt)


def kernel(x, edge_index, W1, film1_W, film1_b, skip1_W, fskip1_W, fskip1_b,
           L1_W, L1_b, W2, film2_W, film2_b, skip2_W, fskip2_W, fskip2_b,
           L2_W, L2_b, W3, film3_W, film3_b, skip3_W, fskip3_W, fskip3_b,
           L3_W, L3_b):
    n = x.shape[0]
    dst0 = edge_index[1]
    perm = jnp.argsort(dst0)
    dst = jnp.take(dst0, perm)
    src = jnp.take(edge_index[0], perm)
    cnt = jax.ops.segment_sum(jnp.ones((src.shape[0],), jnp.float32), dst,
                              num_segments=n, indices_are_sorted=True
                              ).reshape(n, 1)

    def layer(h_in, skip_W, fskip_W, fskip_b, film_W, film_b, lin_W, L_W, L_b,
              out_ch, act):
        base, gamma, beta, hh = _dense_layer(
            h_in, skip_W, fskip_W, fskip_b, film_W, film_b, lin_W, L_W, L_b,
            out_ch)
        msg = _sc_msg(hh, gamma, beta, src, dst, base.shape[1])
        agg = jax.ops.segment_sum(msg, dst, num_segments=n,
                                  indices_are_sorted=True)
        return _combine(base, agg, cnt, act=act)

    h_in = layer(x, skip1_W, fskip1_W, fskip1_b, film1_W, film1_b, W1,
                 L1_W, L1_b, 256, True)
    h_in = layer(h_in, skip2_W, fskip2_W, fskip2_b, film2_W, film2_b, W2,
                 L2_W, L2_b, 256, True)
    out = layer(h_in, skip3_W, fskip3_W, fskip3_b, film3_W, film3_b, W3,
                L3_W, L3_b, 121, False)
    return out[:, :121]


# speedup vs baseline: 1.3260x; 1.3260x over previous
import functools

import jax
import jax.numpy as jnp
from jax import lax
from jax.experimental import pallas as pl
from jax.experimental.pallas import tpu as pltpu
from jax.experimental.pallas import tpu_sc as plsc

N_NODES = 50000
BN = 1000

NC = 2
NS = 16
LANES = 16
E_EDGES = 800000
E_W = E_EDGES // (NC * NS)
K = 64
NB_TOT = (E_W + K - 1) // K
TAIL = E_W - (NB_TOT - 1) * K
NPAIR = (NB_TOT - 1) // 2


def _sc_msg_body(h_hbm, g_hbm, b_hbm, src_hbm, dst_hbm, msg_hbm,
                 src0, dst0, src1, dst1, h0, g0, b0, h1, g1, b1,
                 sem0, sem1, *, C):
    cid = lax.axis_index("c")
    sid = lax.axis_index("s")
    nv = C // LANES
    e0 = (sid * NC + cid) * E_W

    def load_idx(sb, db, b):
        gpos = e0 + b * K
        pltpu.sync_copy(src_hbm.at[pl.ds(gpos, K)], sb)
        pltpu.sync_copy(dst_hbm.at[pl.ds(gpos, K)], db)

    def start(sb, db, hb, gb, bb, sem):
        pltpu.async_copy(h_hbm.at[sb], hb, sem)
        pltpu.async_copy(g_hbm.at[db], gb, sem)
        pltpu.async_copy(b_hbm.at[db], bb, sem)

    def drain(sb, db, hb, gb, bb, sem):
        pltpu.make_async_copy(h_hbm.at[sb], hb, sem).wait()
        pltpu.make_async_copy(g_hbm.at[db], gb, sem).wait()
        pltpu.make_async_copy(b_hbm.at[db], bb, sem).wait()

    def compute(hb, gb, bb):
        def row_body(r, _w):
            for c in range(nv):
                sl = pl.ds(c * LANES, LANES)
                hb[r, sl] = jnp.maximum(gb[r, sl] * hb[r, sl] + bb[r, sl], 0.0)
            return _w
        lax.fori_loop(0, K, row_body, 0)

    def write(hb, b, nreal):
        gpos = e0 + b * K
        if nreal == K:
            pltpu.sync_copy(hb, msg_hbm.at[pl.ds(gpos, K)])
        else:
            pltpu.sync_copy(hb.at[pl.ds(0, nreal)],
                            msg_hbm.at[pl.ds(gpos, nreal)])

    load_idx(src0, dst0, 0)
    start(src0, dst0, h0, g0, b0, sem0)

    def pair_body(j2, _u):
        bb0 = 2 * j2
        load_idx(src1, dst1, bb0 + 1)
        start(src1, dst1, h1, g1, b1, sem1)
        drain(src0, dst0, h0, g0, b0, sem0)
        compute(h0, g0, b0)
        write(h0, bb0, K)
        load_idx(src0, dst0, bb0 + 2)
        start(src0, dst0, h0, g0, b0, sem0)
        drain(src1, dst1, h1, g1, b1, sem1)
        compute(h1, g1, b1)
        write(h1, bb0 + 1, K)
        return _u
    lax.fori_loop(0, NPAIR, pair_body, 0)

    drain(src0, dst0, h0, g0, b0, sem0)
    compute(h0, g0, b0)
    write(h0, NB_TOT - 1, TAIL)


def _sc_msg(h, gamma, beta, src, dst, C):
    mesh = plsc.VectorSubcoreMesh(core_axis_name="c", subcore_axis_name="s")
    kern = pl.kernel(
        functools.partial(_sc_msg_body, C=C),
        out_type=jax.ShapeDtypeStruct((E_EDGES, C), jnp.float32),
        mesh=mesh,
        scratch_types=[
            pltpu.VMEM((K,), jnp.int32),
            pltpu.VMEM((K,), jnp.int32),
            pltpu.VMEM((K,), jnp.int32),
            pltpu.VMEM((K,), jnp.int32),
            pltpu.VMEM((K, C), jnp.float32),
            pltpu.VMEM((K, C), jnp.float32),
            pltpu.VMEM((K, C), jnp.float32),
            pltpu.VMEM((K, C), jnp.float32),
            pltpu.VMEM((K, C), jnp.float32),
            pltpu.VMEM((K, C), jnp.float32),
            pltpu.SemaphoreType.DMA,
            pltpu.SemaphoreType.DMA,
        ],
    )
    return kern(h, gamma, beta, src, dst)


def _dense_body(x_ref, w_ref, b_ref, base_ref, gamma_ref, beta_ref, h_ref, *, cp: int):
    x = x_ref[...]
    z = jnp.dot(x, w_ref[...], preferred_element_type=jnp.float32) + b_ref[...]
    s = z[:, 0:cp]
    fs_beta = z[:, cp:2 * cp]
    fs_gamma = z[:, 2 * cp:3 * cp]
    f_beta = z[:, 3 * cp:4 * cp]
    f_gamma = z[:, 4 * cp:5 * cp]
    lin = z[:, 5 * cp:6 * cp]
    res = z[:, 6 * cp:7 * cp]
    base_ref[...] = jax.nn.relu(fs_gamma * s + fs_beta) + res
    gamma_ref[...] = f_gamma
    beta_ref[...] = f_beta
    h_ref[...] = lin


def _dense_layer(x, skip_W, fskip_W, fskip_b, film_W, film_b, lin_W, L_W, L_b,
                 out_ch):
    n, f = x.shape
    cp = ((out_ch + 127) // 128) * 128
    pad_c = cp - out_ch

    def padc(w):
        return jnp.pad(w, ((0, 0), (0, pad_c))) if pad_c else w

    fs_b = fskip_W[:, :out_ch]
    fs_g = fskip_W[:, out_ch:]
    f_b = film_W[:, :out_ch]
    f_g = film_W[:, out_ch:]
    wcat = jnp.concatenate(
        [padc(skip_W), padc(fs_b), padc(fs_g), padc(f_b), padc(f_g),
         padc(lin_W), padc(L_W)], axis=1)

    def padb(b):
        return jnp.pad(b, (0, pad_c)) if pad_c else b

    zero = jnp.zeros((cp,), jnp.float32)
    bcat = jnp.concatenate(
        [zero, padb(fskip_b[:out_ch]), padb(fskip_b[out_ch:]),
         padb(film_b[:out_ch]), padb(film_b[out_ch:]), zero, padb(L_b)])
    bcat = bcat.reshape(1, 7 * cp)

    fp = ((f + 7) // 8) * 8
    if fp != f:
        x = jnp.pad(x, ((0, 0), (0, fp - f)))
        wcat = jnp.pad(wcat, ((0, fp - f), (0, 0)))

    grid = (n // BN,)
    out_shape = [jax.ShapeDtypeStruct((n, cp), jnp.float32)] * 4
    outs = pl.pallas_call(
        functools.partial(_dense_body, cp=cp),
        grid=grid,
        in_specs=[
            pl.BlockSpec((BN, fp), lambda i: (i, 0)),
            pl.BlockSpec((fp, 7 * cp), lambda i: (0, 0)),
            pl.BlockSpec((1, 7 * cp), lambda i: (0, 0)),
        ],
        out_specs=[pl.BlockSpec((BN, cp), lambda i: (i, 0))] * 4,
        out_shape=out_shape,
    )(x, wcat, bcat)
    return outs


def _combine_body(base_ref, agg_ref, cnt_ref, out_ref, *, act):
    cnt = jnp.maximum(cnt_ref[...], 1.0)
    v = base_ref[...] + agg_ref[...] / cnt
    if act:
        v = jnp.where(v > 0, v, jnp.exp(jnp.minimum(v, 0.0)) - 1.0)
    out_ref[...] = v


def _combine(base, agg_sum, cnt, act):
    n, cp = base.shape
    grid = (n // BN,)
    return pl.pallas_call(
        functools.partial(_combine_body, act=act),
        grid=grid,
        in_specs=[
            pl.BlockSpec((BN, cp), lambda i: (i, 0)),
            pl.BlockSpec((BN, cp), lambda i: (i, 0)),
            pl.BlockSpec((BN, 1), lambda i: (i, 0)),
        ],
        out_specs=pl.BlockSpec((BN, cp), lambda i: (i, 0)),
        out_shape=jax.ShapeDtypeStruct((n, cp), jnp.float32),
    )(base, agg_sum, cnt)


def kernel(x, edge_index, W1, film1_W, film1_b, skip1_W, fskip1_W, fskip1_b,
           L1_W, L1_b, W2, film2_W, film2_b, skip2_W, fskip2_W, fskip2_b,
           L2_W, L2_b, W3, film3_W, film3_b, skip3_W, fskip3_W, fskip3_b,
           L3_W, L3_b):
    n = x.shape[0]
    src = edge_index[0]
    dst = edge_index[1]
    pad = jnp.zeros((K,), jnp.int32)
    src_p = jnp.concatenate([src, pad])
    dst_p = jnp.concatenate([dst, pad])
    cnt = jax.ops.segment_sum(jnp.ones((src.shape[0],), jnp.float32), dst,
                              num_segments=n).reshape(n, 1)

    def layer(h_in, skip_W, fskip_W, fskip_b, film_W, film_b, lin_W, L_W, L_b,
              out_ch, act):
        base, gamma, beta, hh = _dense_layer(
            h_in, skip_W, fskip_W, fskip_b, film_W, film_b, lin_W, L_W, L_b,
            out_ch)
        msg = _sc_msg(hh, gamma, beta, src_p, dst_p, base.shape[1])
        agg = jax.ops.segment_sum(msg, dst, num_segments=n)
        return _combine(base, agg, cnt, act=act)

    h_in = layer(x, skip1_W, fskip1_W, fskip1_b, film1_W, film1_b, W1,
                 L1_W, L1_b, 256, True)
    h_in = layer(h_in, skip2_W, fskip2_W, fskip2_b, film2_W, film2_b, W2,
                 L2_W, L2_b, 256, True)
    out = layer(h_in, skip3_W, fskip3_W, fskip3_b, film3_W, film3_b, W3,
                L3_W, L3_b, 121, False)
    return out[:, :121]
